# baseline (speedup 1.0000x reference)
import functools

import numpy as np
import jax
import jax.numpy as jnp
from jax import lax
from jax.experimental import pallas as pl
from jax.experimental.pallas import tpu as pltpu
from jax.experimental.pallas import tpu_sc as plsc

_N = 10000
_E = 160000
_D = 32
_NB = 10
_RADIUS = 5.0
_MULS = (16, 8, 4)
_NPATH = sum(_MULS)

_NC = 2
_NS = 16
_NW = _NC * _NS

_CH = 128
_EP = 163840
_NSPLIT = 2
_EPS = _EP // _NSPLIT
_EPT = _EPS // _NW
_NCHUNK = _EPT // _CH
_NPAD = _N + 112
_RPT = _NPAD // _NS

_MB = 1024
_NBLK = _EPS // _MB

_SRC_W = 128
_TP_W = 128

@functools.lru_cache(maxsize=None)
def _sc_mesh():
    return plsc.VectorSubcoreMesh(
        core_axis_name="c", subcore_axis_name="s",
        num_cores=_NC, num_subcores=_NS)

_STEP = _RADIUS / float(_NB + 1)
_BC = float(1.14136092 * np.exp(2.0) * np.sqrt(float(_NB)))

_perm = np.empty(_D * _NPATH, np.int32)
_o = 0
_jp = 0
for _l, _mul in zip((0, 1, 2), _MULS):
    for _j in range(_mul):
        for _i in range(_D):
            _perm[_jp * _D + _i] = _o + _i * _mul + _j
        _jp += 1
    _o += _D * _mul

_DIMS = (1, 3, 5)
_G2 = np.zeros((_D * _NPATH, _TP_W), np.float32)
_SSH = np.zeros((9, _TP_W), np.float32)
_c = 0
_jp = 0
for _l, _mul in zip((0, 1, 2), _MULS):
    _dim = _DIMS[_l]
    for _j in range(_mul):
        for _k in range(_dim):
            _G2[_jp * _D:(_jp + 1) * _D, _c] = 1.0
            _SSH[_l * _l + _k, _c] = 1.0
            _c += 1
        _jp += 1


def _silu(v):
    return v / (1.0 + jnp.exp(-v))


def _node_body(x_ref, pos_ref, w1_ref, w2_ref, tsrc_ref):
    h = _silu(jnp.dot(x_ref[...], w1_ref[...], preferred_element_type=jnp.float32))
    emb = jnp.dot(h, w2_ref[...], preferred_element_type=jnp.float32)
    z93 = jnp.zeros((_N, _SRC_W - 35), jnp.float32)
    top = jnp.concatenate([emb, pos_ref[...], z93], axis=1)
    tsrc_ref[...] = jnp.concatenate(
        [top, jnp.zeros((_NPAD - _N, _SRC_W), jnp.float32)], axis=0)


_node_call = pl.pallas_call(
    _node_body,
    out_shape=jax.ShapeDtypeStruct((_NPAD, _SRC_W), jnp.float32),
)


def _gather_body(src_hbm, dst_hbm, tsrc_hbm, xe_out,
                 idxs_v, idxd_v, rs_v, rd_v, sems, semd):
    wid = lax.axis_index("s") * _NC + lax.axis_index("c")
    base0 = wid * _EPT

    def body(j, carry):
        b = base0 + j * _CH
        pltpu.sync_copy(src_hbm.at[pl.ds(b, _CH)], idxs_v)
        pltpu.sync_copy(dst_hbm.at[pl.ds(b, _CH)], idxd_v)
        cs = pltpu.async_copy(tsrc_hbm.at[idxs_v], rs_v, sems)
        cd = pltpu.async_copy(tsrc_hbm.at[idxd_v], rd_v, semd)
        cs.wait()
        cd.wait()

        def evbody(e, c2):
            a = rs_v[e, pl.ds(32, 16)]
            bvec = rd_v[e, pl.ds(32, 16)]
            rs_v[e, pl.ds(32, 16)] = bvec - a
            return c2

        lax.fori_loop(0, _CH, evbody, 0)
        pltpu.sync_copy(rs_v, xe_out.at[pl.ds(b, _CH)])
        return carry

    lax.fori_loop(0, _NCHUNK, body, 0)


@functools.lru_cache(maxsize=None)
def _gather_call():
    return pl.kernel(
        _gather_body,
        out_type=jax.ShapeDtypeStruct((_EPS, _SRC_W), jnp.float32),
        mesh=_sc_mesh(),
        scratch_types=[
            pltpu.VMEM((_CH,), jnp.int32),
            pltpu.VMEM((_CH,), jnp.int32),
            pltpu.VMEM((_CH, _SRC_W), jnp.float32),
            pltpu.VMEM((_CH, _SRC_W), jnp.float32),
            pltpu.SemaphoreType.DMA,
            pltpu.SemaphoreType.DMA,
        ],
    )


def _edge_body(xepos_ref, v1_ref, v2_ref, g2_ref, ssh_ref, tp_ref):
    xep = xepos_ref[...]
    xe = xep[:, 0:32]
    ev = xep[:, 32:48]
    r2 = jnp.sum(ev * ev, axis=1, keepdims=True) + 1e-12
    r = jnp.sqrt(r2)
    u = ev[:, 0:3] / r
    ux, uy, uz = u[:, 0:1], u[:, 1:2], u[:, 2:3]
    s3 = np.sqrt(3.0)
    s15 = np.sqrt(15.0)
    s5 = np.sqrt(5.0)
    sh1 = jnp.concatenate([s3 * uy, s3 * uz, s3 * ux], axis=1)
    sh2 = jnp.concatenate([
        s15 * ux * uy, s15 * uy * uz, 0.5 * s5 * (3.0 * uz * uz - 1.0),
        s15 * uz * ux, 0.5 * s15 * (ux * ux - uy * uy)], axis=1)

    k16 = lax.broadcasted_iota(jnp.int32, (1, 16), 1).astype(jnp.float32)
    vals16 = jnp.where(k16 < float(_NB), (k16 + 1.0) * _STEP, 1e9)
    ub = (r - vals16) / _STEP
    inside = jnp.abs(ub) < 1.0
    den = jnp.where(inside, 1.0 - ub * ub, 1.0)
    basis = jnp.where(inside, _BC * jnp.exp(-1.0 / den), 0.0)

    h = _silu(jnp.dot(basis, v1_ref[...], preferred_element_type=jnp.float32))
    w = jnp.dot(h, v2_ref[...], preferred_element_type=jnp.float32)

    xt = jnp.concatenate([xe] * _NPATH, axis=1)
    pg = jnp.dot(w * xt, g2_ref[...], preferred_element_type=jnp.float32)
    sh9 = jnp.concatenate([jnp.ones((_MB, 1), jnp.float32), sh1, sh2], axis=1)
    shx = jnp.dot(sh9, ssh_ref[...], preferred_element_type=jnp.float32)
    tp_ref[...] = pg * shx


_edge_call = pl.pallas_call(
    _edge_body,
    grid=(_NBLK,),
    in_specs=[
        pl.BlockSpec((_MB, _SRC_W), lambda i: (i, 0)),
        pl.BlockSpec((16, 64), lambda i: (0, 0)),
        pl.BlockSpec((64, _D * _NPATH), lambda i: (0, 0)),
        pl.BlockSpec((_D * _NPATH, _TP_W), lambda i: (0, 0)),
        pl.BlockSpec((9, _TP_W), lambda i: (0, 0)),
    ],
    out_specs=pl.BlockSpec((_MB, _TP_W), lambda i: (i, 0)),
    out_shape=jax.ShapeDtypeStruct((_EPS, _TP_W), jnp.float32),
)


def _scatter_body(dst_hbm, tp_hbm, zero_hbm, out_hbm, idx_v, rows_v, acc_sh, sem):
    cid = lax.axis_index("c")
    sid = lax.axis_index("s")
    rb = sid * _RPT
    pltpu.sync_copy(zero_hbm.at[pl.ds(rb, _RPT)], acc_sh.at[pl.ds(rb, _RPT)])
    plsc.subcore_barrier()

    ebase = cid * (_EPS // _NC) + sid * _EPT

    def body(j, carry):
        b = ebase + j * _CH
        pltpu.sync_copy(dst_hbm.at[pl.ds(b, _CH)], idx_v)
        pltpu.sync_copy(tp_hbm.at[pl.ds(b, _CH)], rows_v)
        pltpu.sync_copy(rows_v, acc_sh.at[idx_v], add=True)
        return carry

    lax.fori_loop(0, _NCHUNK, body, 0)
    plsc.subcore_barrier()
    pltpu.sync_copy(acc_sh.at[pl.ds(rb, _RPT)],
                    out_hbm.at[pl.ds(cid * _NPAD + rb, _RPT)])


@functools.lru_cache(maxsize=None)
def _scatter_call():
    return pl.kernel(
        _scatter_body,
        out_type=jax.ShapeDtypeStruct((_NC * _NPAD, _TP_W), jnp.float32),
        mesh=_sc_mesh(),
        scratch_types=[
            pltpu.VMEM((_CH,), jnp.int32),
            pltpu.VMEM((_CH, _TP_W), jnp.float32),
            pltpu.VMEM_SHARED((_NPAD, _TP_W), jnp.float32),
            pltpu.SemaphoreType.DMA,
        ],
    )


def _combine_body(tsrc_ref, p0_ref, p1_ref, out_ref):
    emb = tsrc_ref[pl.ds(0, _N), 0:32]
    m = (p0_ref[pl.ds(0, _N), :] + p0_ref[pl.ds(_NPAD, _N), :]
         + p1_ref[pl.ds(0, _N), :] + p1_ref[pl.ds(_NPAD, _N), :])
    out_ref[...] = jnp.concatenate([emb, m[:, 0:60]], axis=1)


_combine_call = pl.pallas_call(
    _combine_body,
    out_shape=jax.ShapeDtypeStruct((_N, 92), jnp.float32),
)


def kernel(x, pos, edge_index, W1, W2, V1, V2):
    w1s = W1 * np.float32(1.0 / np.sqrt(32.0))
    w2s = W2 * np.float32(1.0 / np.sqrt(128.0))
    v1s = jnp.concatenate(
        [V1 * np.float32(1.0 / np.sqrt(10.0)),
         jnp.zeros((6, 64), jnp.float32)], axis=0)
    scale = np.float32(1.0 / (np.sqrt(64.0) * np.sqrt(32.0) * np.sqrt(16.0)))
    v2jm = (V2 * scale)[:, _perm]

    src = edge_index[0].astype(jnp.int32)
    dst = edge_index[1].astype(jnp.int32)
    src_p = jnp.concatenate([src, jnp.zeros((_EP - _E,), jnp.int32)])
    dst_p = jnp.concatenate([dst, jnp.full((_EP - _E,), _N, jnp.int32)])
    zeros = jnp.zeros((_NPAD, _TP_W), jnp.float32)

    tsrc = _node_call(x, pos, w1s, w2s)
    parts = []
    for s in range(_NSPLIT):
        sl = slice(s * _EPS, (s + 1) * _EPS)
        xepos = _gather_call()(src_p[sl], dst_p[sl], tsrc)
        tp = _edge_call(xepos, v1s, v2jm, _G2, _SSH)
        parts.append(_scatter_call()(dst_p[sl], tp, zeros))
    return _combine_call(tsrc, parts[0], parts[1])


# speedup vs baseline: 1.1335x; 1.1335x over previous
import functools

import numpy as np
import jax
import jax.numpy as jnp
from jax import lax
from jax.experimental import pallas as pl
from jax.experimental.pallas import tpu as pltpu
from jax.experimental.pallas import tpu_sc as plsc

_N = 10000
_E = 160000
_D = 32
_NB = 10
_RADIUS = 5.0
_MULS = (16, 8, 4)
_NPATH = sum(_MULS)

_NC = 2
_NS = 16
_NW = _NC * _NS

_CH = 128
_EP = 163840
_NSPLIT = 2
_EPS = _EP // _NSPLIT
_EPT = _EPS // _NW
_NCHUNK = _EPT // _CH
_NPAD = _N + 112
_RPT = _NPAD // _NS

_MB = 1024
_NBLK = _EPS // _MB

_SRC_W = 48
_DST_W = 16
_TP_W = 128

@functools.lru_cache(maxsize=None)
def _sc_mesh():
    return plsc.VectorSubcoreMesh(
        core_axis_name="c", subcore_axis_name="s",
        num_cores=_NC, num_subcores=_NS)

_STEP = _RADIUS / float(_NB + 1)
_BC = float(1.14136092 * np.exp(2.0) * np.sqrt(float(_NB)))

_perm = np.empty(_D * _NPATH, np.int32)
_o = 0
_jp = 0
for _l, _mul in zip((0, 1, 2), _MULS):
    for _j in range(_mul):
        for _i in range(_D):
            _perm[_jp * _D + _i] = _o + _i * _mul + _j
        _jp += 1
    _o += _D * _mul

_DIMS = (1, 3, 5)
_G2 = np.zeros((_D * _NPATH, _TP_W), np.float32)
_SSH = np.zeros((9, _TP_W), np.float32)
_c = 0
_jp = 0
for _l, _mul in zip((0, 1, 2), _MULS):
    _dim = _DIMS[_l]
    for _j in range(_mul):
        for _k in range(_dim):
            _G2[_jp * _D:(_jp + 1) * _D, _c] = 1.0
            _SSH[_l * _l + _k, _c] = 1.0
            _c += 1
        _jp += 1


def _silu(v):
    return v / (1.0 + jnp.exp(-v))


def _node_body(x_ref, pos_ref, w1_ref, w2_ref, tsrc_ref, tdst_ref):
    h = _silu(jnp.dot(x_ref[...], w1_ref[...], preferred_element_type=jnp.float32))
    emb = jnp.dot(h, w2_ref[...], preferred_element_type=jnp.float32)
    z13 = jnp.zeros((_N, _SRC_W - 35), jnp.float32)
    tsrc_ref[...] = jnp.concatenate([emb, pos_ref[...], z13], axis=1)
    pd_top = jnp.concatenate(
        [pos_ref[...], jnp.zeros((_N, _DST_W - 3), jnp.float32)], axis=1)
    tdst_ref[...] = jnp.concatenate(
        [pd_top, jnp.zeros((_NPAD - _N, _DST_W), jnp.float32)], axis=0)


_node_call = pl.pallas_call(
    _node_body,
    out_shape=[
        jax.ShapeDtypeStruct((_N, _SRC_W), jnp.float32),
        jax.ShapeDtypeStruct((_NPAD, _DST_W), jnp.float32),
    ],
)


def _gather_body(src_hbm, dst_hbm, tsrc_hbm, tdst_hbm, xe_out,
                 idxs_v, idxd_v, rs_v, rd_v, semgs, semgd, semw):
    wid = lax.axis_index("s") * _NC + lax.axis_index("c")
    base0 = wid * _EPT

    pltpu.sync_copy(src_hbm.at[pl.ds(base0, _CH)], idxs_v.at[0])
    pltpu.sync_copy(dst_hbm.at[pl.ds(base0, _CH)], idxd_v.at[0])
    pltpu.async_copy(tsrc_hbm.at[idxs_v.at[0]], rs_v.at[0], semgs)
    pltpu.async_copy(tdst_hbm.at[idxd_v.at[0]], rd_v.at[0], semgd)

    def body(j, carry):
        cur = j % 2
        nxt = (j + 1) % 2
        b = base0 + j * _CH

        @pl.when(j < _NCHUNK - 1)
        def _prefetch_idx():
            b2 = b + _CH
            pltpu.sync_copy(src_hbm.at[pl.ds(b2, _CH)], idxs_v.at[nxt])
            pltpu.sync_copy(dst_hbm.at[pl.ds(b2, _CH)], idxd_v.at[nxt])

        pltpu.make_async_copy(tsrc_hbm.at[idxs_v.at[cur]], rs_v.at[cur], semgs).wait()
        pltpu.make_async_copy(tdst_hbm.at[idxd_v.at[cur]], rd_v.at[cur], semgd).wait()

        @pl.when(j > 0)
        def _drain_writeback():
            pltpu.make_async_copy(
                rs_v.at[nxt], xe_out.at[pl.ds(b - _CH, _CH)], semw).wait()

        @pl.when(j < _NCHUNK - 1)
        def _launch_next():
            pltpu.async_copy(tsrc_hbm.at[idxs_v.at[nxt]], rs_v.at[nxt], semgs)
            pltpu.async_copy(tdst_hbm.at[idxd_v.at[nxt]], rd_v.at[nxt], semgd)

        def evbody(e, c2):
            a = rs_v[cur, e, pl.ds(32, 16)]
            bvec = rd_v[cur, e, pl.ds(0, 16)]
            rs_v[cur, e, pl.ds(32, 16)] = bvec - a
            return c2

        lax.fori_loop(0, _CH, evbody, 0)
        pltpu.async_copy(rs_v.at[cur], xe_out.at[pl.ds(b, _CH)], semw)
        return carry

    lax.fori_loop(0, _NCHUNK, body, 0)
    last = (_NCHUNK - 1) % 2
    pltpu.make_async_copy(
        rs_v.at[last],
        xe_out.at[pl.ds(base0 + (_NCHUNK - 1) * _CH, _CH)], semw).wait()


@functools.lru_cache(maxsize=None)
def _gather_call():
    return pl.kernel(
        _gather_body,
        out_type=jax.ShapeDtypeStruct((_EPS, _SRC_W), jnp.float32),
        mesh=_sc_mesh(),
        scratch_types=[
            pltpu.VMEM((2, _CH), jnp.int32),
            pltpu.VMEM((2, _CH), jnp.int32),
            pltpu.VMEM((2, _CH, _SRC_W), jnp.float32),
            pltpu.VMEM((2, _CH, _DST_W), jnp.float32),
            pltpu.SemaphoreType.DMA,
            pltpu.SemaphoreType.DMA,
            pltpu.SemaphoreType.DMA,
        ],
        compiler_params=pltpu.CompilerParams(use_tc_tiling_on_sc=False),
    )


def _edge_body(xepos_ref, v1_ref, v2_ref, g2_ref, ssh_ref, tp_ref):
    xep = xepos_ref[...]
    xe = xep[:, 0:32]
    ev = xep[:, 32:48]
    r2 = jnp.sum(ev * ev, axis=1, keepdims=True) + 1e-12
    r = jnp.sqrt(r2)
    u = ev[:, 0:3] / r
    ux, uy, uz = u[:, 0:1], u[:, 1:2], u[:, 2:3]
    s3 = np.sqrt(3.0)
    s15 = np.sqrt(15.0)
    s5 = np.sqrt(5.0)
    sh1 = jnp.concatenate([s3 * uy, s3 * uz, s3 * ux], axis=1)
    sh2 = jnp.concatenate([
        s15 * ux * uy, s15 * uy * uz, 0.5 * s5 * (3.0 * uz * uz - 1.0),
        s15 * uz * ux, 0.5 * s15 * (ux * ux - uy * uy)], axis=1)

    k16 = lax.broadcasted_iota(jnp.int32, (1, 16), 1).astype(jnp.float32)
    vals16 = jnp.where(k16 < float(_NB), (k16 + 1.0) * _STEP, 1e9)
    ub = (r - vals16) / _STEP
    inside = jnp.abs(ub) < 1.0
    den = jnp.where(inside, 1.0 - ub * ub, 1.0)
    basis = jnp.where(inside, _BC * jnp.exp(-1.0 / den), 0.0)

    h = _silu(jnp.dot(basis, v1_ref[...], preferred_element_type=jnp.float32))
    w = jnp.dot(h, v2_ref[...], preferred_element_type=jnp.float32)

    xt = jnp.concatenate([xe] * _NPATH, axis=1)
    pg = jnp.dot(w * xt, g2_ref[...], preferred_element_type=jnp.float32)
    sh9 = jnp.concatenate([jnp.ones((_MB, 1), jnp.float32), sh1, sh2], axis=1)
    shx = jnp.dot(sh9, ssh_ref[...], preferred_element_type=jnp.float32)
    tp_ref[...] = pg * shx


_edge_call = pl.pallas_call(
    _edge_body,
    grid=(_NBLK,),
    in_specs=[
        pl.BlockSpec((_MB, _SRC_W), lambda i: (i, 0)),
        pl.BlockSpec((16, 64), lambda i: (0, 0)),
        pl.BlockSpec((64, _D * _NPATH), lambda i: (0, 0)),
        pl.BlockSpec((_D * _NPATH, _TP_W), lambda i: (0, 0)),
        pl.BlockSpec((9, _TP_W), lambda i: (0, 0)),
    ],
    out_specs=pl.BlockSpec((_MB, _TP_W), lambda i: (i, 0)),
    out_shape=jax.ShapeDtypeStruct((_EPS, _TP_W), jnp.float32),
)


def _scatter_body(dst_hbm, tp_hbm, zero_hbm, out_hbm, idx_v, rows_v, acc_sh,
                  semi, semr, sema):
    cid = lax.axis_index("c")
    sid = lax.axis_index("s")
    rb = sid * _RPT
    ebase = cid * (_EPS // _NC) + sid * _EPT

    pltpu.async_copy(dst_hbm.at[pl.ds(ebase, _CH)], idx_v.at[0], semi)
    pltpu.async_copy(tp_hbm.at[pl.ds(ebase, _CH)], rows_v.at[0], semr)
    pltpu.sync_copy(zero_hbm.at[pl.ds(rb, _RPT)], acc_sh.at[pl.ds(rb, _RPT)])
    plsc.subcore_barrier()

    def body(j, carry):
        cur = j % 2
        nxt = (j + 1) % 2
        b = ebase + j * _CH
        pltpu.make_async_copy(dst_hbm.at[pl.ds(b, _CH)], idx_v.at[cur], semi).wait()
        pltpu.make_async_copy(tp_hbm.at[pl.ds(b, _CH)], rows_v.at[cur], semr).wait()

        @pl.when(j > 0)
        def _drain_add():
            pltpu.make_async_copy(
                rows_v.at[nxt], acc_sh.at[idx_v.at[nxt]], sema).wait()

        @pl.when(j < _NCHUNK - 1)
        def _launch_next():
            b2 = b + _CH
            pltpu.async_copy(dst_hbm.at[pl.ds(b2, _CH)], idx_v.at[nxt], semi)
            pltpu.async_copy(tp_hbm.at[pl.ds(b2, _CH)], rows_v.at[nxt], semr)

        pltpu.async_copy(rows_v.at[cur], acc_sh.at[idx_v.at[cur]], sema, add=True)
        return carry

    lax.fori_loop(0, _NCHUNK, body, 0)
    last = (_NCHUNK - 1) % 2
    pltpu.make_async_copy(rows_v.at[last], acc_sh.at[idx_v.at[last]], sema).wait()
    plsc.subcore_barrier()
    pltpu.sync_copy(acc_sh.at[pl.ds(rb, _RPT)],
                    out_hbm.at[pl.ds(cid * _NPAD + rb, _RPT)])


@functools.lru_cache(maxsize=None)
def _scatter_call():
    return pl.kernel(
        _scatter_body,
        out_type=jax.ShapeDtypeStruct((_NC * _NPAD, _TP_W), jnp.float32),
        mesh=_sc_mesh(),
        scratch_types=[
            pltpu.VMEM((2, _CH), jnp.int32),
            pltpu.VMEM((2, _CH, _TP_W), jnp.float32),
            pltpu.VMEM_SHARED((_NPAD, _TP_W), jnp.float32),
            pltpu.SemaphoreType.DMA,
            pltpu.SemaphoreType.DMA,
            pltpu.SemaphoreType.DMA,
        ],
    )


def _combine_body(tsrc_ref, p0_ref, p1_ref, out_ref):
    emb = tsrc_ref[...][:, 0:32]
    m = (p0_ref[pl.ds(0, _N), :] + p0_ref[pl.ds(_NPAD, _N), :]
         + p1_ref[pl.ds(0, _N), :] + p1_ref[pl.ds(_NPAD, _N), :])
    out_ref[...] = jnp.concatenate([emb, m[:, 0:60]], axis=1)


_combine_call = pl.pallas_call(
    _combine_body,
    out_shape=jax.ShapeDtypeStruct((_N, 92), jnp.float32),
)


def kernel(x, pos, edge_index, W1, W2, V1, V2):
    w1s = W1 * np.float32(1.0 / np.sqrt(32.0))
    w2s = W2 * np.float32(1.0 / np.sqrt(128.0))
    v1s = jnp.concatenate(
        [V1 * np.float32(1.0 / np.sqrt(10.0)),
         jnp.zeros((6, 64), jnp.float32)], axis=0)
    scale = np.float32(1.0 / (np.sqrt(64.0) * np.sqrt(32.0) * np.sqrt(16.0)))
    v2jm = (V2 * scale)[:, _perm]

    src = edge_index[0].astype(jnp.int32)
    dst = edge_index[1].astype(jnp.int32)
    src_p = jnp.concatenate([src, jnp.zeros((_EP - _E,), jnp.int32)])
    dst_p = jnp.concatenate([dst, jnp.full((_EP - _E,), _N, jnp.int32)])
    zeros = jnp.zeros((_NPAD, _TP_W), jnp.float32)

    tsrc, tdst = _node_call(x, pos, w1s, w2s)
    parts = []
    for s in range(_NSPLIT):
        sl = slice(s * _EPS, (s + 1) * _EPS)
        xepos = _gather_call()(src_p[sl], dst_p[sl], tsrc, tdst)
        tp = _edge_call(xepos, v1s, v2jm, _G2, _SSH)
        parts.append(_scatter_call()(dst_p[sl], tp, zeros))
    return _combine_call(tsrc, parts[0], parts[1])


# speedup vs baseline: 1.2541x; 1.1063x over previous
import functools

import numpy as np
import jax
import jax.numpy as jnp
from jax import lax
from jax.experimental import pallas as pl
from jax.experimental.pallas import tpu as pltpu
from jax.experimental.pallas import tpu_sc as plsc

_N = 10000
_E = 160000
_D = 32
_NB = 10
_RADIUS = 5.0
_MULS = (16, 8, 4)
_NPATH = sum(_MULS)

_NC = 2
_NS = 16
_NW = _NC * _NS

_CH = 128
_EP = 163840
_NSPLIT = 4
_EPS = _EP // _NSPLIT
_EPT = _EPS // _NW
_NCHUNK = _EPT // _CH
_NPAD = _N + 112
_RPT = _NPAD // _NS

_MB = 1024
_NBLK = _EPS // _MB

_SRC_W = 48
_DST_W = 16
_TP_W = 128

@functools.lru_cache(maxsize=None)
def _sc_mesh():
    return plsc.VectorSubcoreMesh(
        core_axis_name="c", subcore_axis_name="s",
        num_cores=_NC, num_subcores=_NS)

_STEP = _RADIUS / float(_NB + 1)
_BC = float(1.14136092 * np.exp(2.0) * np.sqrt(float(_NB)))

_perm = np.empty(_D * _NPATH, np.int32)
_o = 0
_jp = 0
for _l, _mul in zip((0, 1, 2), _MULS):
    for _j in range(_mul):
        for _i in range(_D):
            _perm[_jp * _D + _i] = _o + _i * _mul + _j
        _jp += 1
    _o += _D * _mul

_DIMS = (1, 3, 5)
_G2 = np.zeros((_D * _NPATH, _TP_W), np.float32)
_SSH = np.zeros((9, _TP_W), np.float32)
_c = 0
_jp = 0
for _l, _mul in zip((0, 1, 2), _MULS):
    _dim = _DIMS[_l]
    for _j in range(_mul):
        for _k in range(_dim):
            _G2[_jp * _D:(_jp + 1) * _D, _c] = 1.0
            _SSH[_l * _l + _k, _c] = 1.0
            _c += 1
        _jp += 1


def _silu(v):
    return v / (1.0 + jnp.exp(-v))


def _node_body(x_ref, pos_ref, w1_ref, w2_ref, tsrc_ref, tdst_ref):
    h = _silu(jnp.dot(x_ref[...], w1_ref[...], preferred_element_type=jnp.float32))
    emb = jnp.dot(h, w2_ref[...], preferred_element_type=jnp.float32)
    z13 = jnp.zeros((_N, _SRC_W - 35), jnp.float32)
    tsrc_ref[...] = jnp.concatenate([emb, pos_ref[...], z13], axis=1)
    pd_top = jnp.concatenate(
        [pos_ref[...], jnp.zeros((_N, _DST_W - 3), jnp.float32)], axis=1)
    tdst_ref[...] = jnp.concatenate(
        [pd_top, jnp.zeros((_NPAD - _N, _DST_W), jnp.float32)], axis=0)


_node_call = pl.pallas_call(
    _node_body,
    out_shape=[
        jax.ShapeDtypeStruct((_N, _SRC_W), jnp.float32),
        jax.ShapeDtypeStruct((_NPAD, _DST_W), jnp.float32),
    ],
)


def _gather_body(src_hbm, dst_hbm, tsrc_hbm, tdst_hbm, xe_out,
                 idxs_v, idxd_v, rs_v, rd_v, semgs, semgd, semw):
    wid = lax.axis_index("s") * _NC + lax.axis_index("c")
    base0 = wid * _EPT

    pltpu.sync_copy(src_hbm.at[pl.ds(base0, _CH)], idxs_v.at[0])
    pltpu.sync_copy(dst_hbm.at[pl.ds(base0, _CH)], idxd_v.at[0])
    pltpu.async_copy(tsrc_hbm.at[idxs_v.at[0]], rs_v.at[0], semgs)
    pltpu.async_copy(tdst_hbm.at[idxd_v.at[0]], rd_v.at[0], semgd)

    def body(j, carry):
        cur = j % 2
        nxt = (j + 1) % 2
        b = base0 + j * _CH

        @pl.when(j < _NCHUNK - 1)
        def _prefetch_idx():
            b2 = b + _CH
            pltpu.sync_copy(src_hbm.at[pl.ds(b2, _CH)], idxs_v.at[nxt])
            pltpu.sync_copy(dst_hbm.at[pl.ds(b2, _CH)], idxd_v.at[nxt])

        pltpu.make_async_copy(tsrc_hbm.at[idxs_v.at[cur]], rs_v.at[cur], semgs).wait()
        pltpu.make_async_copy(tdst_hbm.at[idxd_v.at[cur]], rd_v.at[cur], semgd).wait()

        @pl.when(j > 0)
        def _drain_writeback():
            pltpu.make_async_copy(
                rs_v.at[nxt], xe_out.at[pl.ds(b - _CH, _CH)], semw).wait()

        @pl.when(j < _NCHUNK - 1)
        def _launch_next():
            pltpu.async_copy(tsrc_hbm.at[idxs_v.at[nxt]], rs_v.at[nxt], semgs)
            pltpu.async_copy(tdst_hbm.at[idxd_v.at[nxt]], rd_v.at[nxt], semgd)

        def evbody(e, c2):
            a = rs_v[cur, e, pl.ds(32, 16)]
            bvec = rd_v[cur, e, pl.ds(0, 16)]
            rs_v[cur, e, pl.ds(32, 16)] = bvec - a
            return c2

        lax.fori_loop(0, _CH, evbody, 0)
        pltpu.async_copy(rs_v.at[cur], xe_out.at[pl.ds(b, _CH)], semw)
        return carry

    lax.fori_loop(0, _NCHUNK, body, 0)
    last = (_NCHUNK - 1) % 2
    pltpu.make_async_copy(
        rs_v.at[last],
        xe_out.at[pl.ds(base0 + (_NCHUNK - 1) * _CH, _CH)], semw).wait()


@functools.lru_cache(maxsize=None)
def _gather_call():
    return pl.kernel(
        _gather_body,
        out_type=jax.ShapeDtypeStruct((_EPS, _SRC_W), jnp.float32),
        mesh=_sc_mesh(),
        scratch_types=[
            pltpu.VMEM((2, _CH), jnp.int32),
            pltpu.VMEM((2, _CH), jnp.int32),
            pltpu.VMEM((2, _CH, _SRC_W), jnp.float32),
            pltpu.VMEM((2, _CH, _DST_W), jnp.float32),
            pltpu.SemaphoreType.DMA,
            pltpu.SemaphoreType.DMA,
            pltpu.SemaphoreType.DMA,
        ],
        compiler_params=pltpu.CompilerParams(use_tc_tiling_on_sc=False),
    )


def _edge_body(xepos_ref, v1_ref, v2_ref, g2_ref, ssh_ref, tp_ref):
    xep = xepos_ref[...]
    xe = xep[:, 0:32]
    ev = xep[:, 32:48]
    r2 = jnp.sum(ev * ev, axis=1, keepdims=True) + 1e-12
    r = jnp.sqrt(r2)
    u = ev[:, 0:3] / r
    ux, uy, uz = u[:, 0:1], u[:, 1:2], u[:, 2:3]
    s3 = np.sqrt(3.0)
    s15 = np.sqrt(15.0)
    s5 = np.sqrt(5.0)
    sh1 = jnp.concatenate([s3 * uy, s3 * uz, s3 * ux], axis=1)
    sh2 = jnp.concatenate([
        s15 * ux * uy, s15 * uy * uz, 0.5 * s5 * (3.0 * uz * uz - 1.0),
        s15 * uz * ux, 0.5 * s15 * (ux * ux - uy * uy)], axis=1)

    k16 = lax.broadcasted_iota(jnp.int32, (1, 16), 1).astype(jnp.float32)
    vals16 = jnp.where(k16 < float(_NB), (k16 + 1.0) * _STEP, 1e9)
    ub = (r - vals16) / _STEP
    inside = jnp.abs(ub) < 1.0
    den = jnp.where(inside, 1.0 - ub * ub, 1.0)
    basis = jnp.where(inside, _BC * jnp.exp(-1.0 / den), 0.0)

    h = _silu(jnp.dot(basis, v1_ref[...], preferred_element_type=jnp.float32))
    w = jnp.dot(h, v2_ref[...], preferred_element_type=jnp.float32)

    xt = jnp.concatenate([xe] * _NPATH, axis=1)
    pg = jnp.dot(w * xt, g2_ref[...], preferred_element_type=jnp.float32)
    sh9 = jnp.concatenate([jnp.ones((_MB, 1), jnp.float32), sh1, sh2], axis=1)
    shx = jnp.dot(sh9, ssh_ref[...], preferred_element_type=jnp.float32)
    tp_ref[...] = pg * shx


_edge_call = pl.pallas_call(
    _edge_body,
    grid=(_NBLK,),
    in_specs=[
        pl.BlockSpec((_MB, _SRC_W), lambda i: (i, 0)),
        pl.BlockSpec((16, 64), lambda i: (0, 0)),
        pl.BlockSpec((64, _D * _NPATH), lambda i: (0, 0)),
        pl.BlockSpec((_D * _NPATH, _TP_W), lambda i: (0, 0)),
        pl.BlockSpec((9, _TP_W), lambda i: (0, 0)),
    ],
    out_specs=pl.BlockSpec((_MB, _TP_W), lambda i: (i, 0)),
    out_shape=jax.ShapeDtypeStruct((_EPS, _TP_W), jnp.float32),
)


def _scatter_body(dst_hbm, tp_hbm, zero_hbm, out_hbm, idx_v, rows_v, acc_sh,
                  semi, semr, sema):
    cid = lax.axis_index("c")
    sid = lax.axis_index("s")
    rb = sid * _RPT
    ebase = cid * (_EPS // _NC) + sid * _EPT

    pltpu.async_copy(dst_hbm.at[pl.ds(ebase, _CH)], idx_v.at[0], semi)
    pltpu.async_copy(tp_hbm.at[pl.ds(ebase, _CH)], rows_v.at[0], semr)
    pltpu.sync_copy(zero_hbm.at[pl.ds(rb, _RPT)], acc_sh.at[pl.ds(rb, _RPT)])
    plsc.subcore_barrier()

    def body(j, carry):
        cur = j % 2
        nxt = (j + 1) % 2
        b = ebase + j * _CH
        pltpu.make_async_copy(dst_hbm.at[pl.ds(b, _CH)], idx_v.at[cur], semi).wait()
        pltpu.make_async_copy(tp_hbm.at[pl.ds(b, _CH)], rows_v.at[cur], semr).wait()

        @pl.when(j > 0)
        def _drain_add():
            pltpu.make_async_copy(
                rows_v.at[nxt], acc_sh.at[idx_v.at[nxt]], sema).wait()

        @pl.when(j < _NCHUNK - 1)
        def _launch_next():
            b2 = b + _CH
            pltpu.async_copy(dst_hbm.at[pl.ds(b2, _CH)], idx_v.at[nxt], semi)
            pltpu.async_copy(tp_hbm.at[pl.ds(b2, _CH)], rows_v.at[nxt], semr)

        pltpu.async_copy(rows_v.at[cur], acc_sh.at[idx_v.at[cur]], sema, add=True)
        return carry

    lax.fori_loop(0, _NCHUNK, body, 0)
    last = (_NCHUNK - 1) % 2
    pltpu.make_async_copy(rows_v.at[last], acc_sh.at[idx_v.at[last]], sema).wait()
    plsc.subcore_barrier()
    pltpu.sync_copy(acc_sh.at[pl.ds(rb, _RPT)],
                    out_hbm.at[pl.ds(cid * _NPAD + rb, _RPT)])


@functools.lru_cache(maxsize=None)
def _scatter_call():
    return pl.kernel(
        _scatter_body,
        out_type=jax.ShapeDtypeStruct((_NC * _NPAD, _TP_W), jnp.float32),
        mesh=_sc_mesh(),
        scratch_types=[
            pltpu.VMEM((2, _CH), jnp.int32),
            pltpu.VMEM((2, _CH, _TP_W), jnp.float32),
            pltpu.VMEM_SHARED((_NPAD, _TP_W), jnp.float32),
            pltpu.SemaphoreType.DMA,
            pltpu.SemaphoreType.DMA,
            pltpu.SemaphoreType.DMA,
        ],
    )


def _combine_body(tsrc_ref, *refs):
    part_refs = refs[:_NSPLIT]
    out_ref = refs[_NSPLIT]
    emb = tsrc_ref[...][:, 0:32]
    m = part_refs[0][pl.ds(0, _N), :] + part_refs[0][pl.ds(_NPAD, _N), :]
    for p in part_refs[1:]:
        m = m + p[pl.ds(0, _N), :] + p[pl.ds(_NPAD, _N), :]
    out_ref[...] = jnp.concatenate([emb, m[:, 0:60]], axis=1)


_combine_call = pl.pallas_call(
    _combine_body,
    out_shape=jax.ShapeDtypeStruct((_N, 92), jnp.float32),
)


def kernel(x, pos, edge_index, W1, W2, V1, V2):
    w1s = W1 * np.float32(1.0 / np.sqrt(32.0))
    w2s = W2 * np.float32(1.0 / np.sqrt(128.0))
    v1s = jnp.concatenate(
        [V1 * np.float32(1.0 / np.sqrt(10.0)),
         jnp.zeros((6, 64), jnp.float32)], axis=0)
    scale = np.float32(1.0 / (np.sqrt(64.0) * np.sqrt(32.0) * np.sqrt(16.0)))
    v2jm = (V2 * scale)[:, _perm]

    src = edge_index[0].astype(jnp.int32)
    dst = edge_index[1].astype(jnp.int32)
    src_p = jnp.concatenate([src, jnp.zeros((_EP - _E,), jnp.int32)])
    dst_p = jnp.concatenate([dst, jnp.full((_EP - _E,), _N, jnp.int32)])
    zeros = jnp.zeros((_NPAD, _TP_W), jnp.float32)

    tsrc, tdst = _node_call(x, pos, w1s, w2s)
    parts = []
    for s in range(_NSPLIT):
        sl = slice(s * _EPS, (s + 1) * _EPS)
        xepos = _gather_call()(src_p[sl], dst_p[sl], tsrc, tdst)
        tp = _edge_call(xepos, v1s, v2jm, _G2, _SSH)
        parts.append(_scatter_call()(dst_p[sl], tp, zeros))
    return _combine_call(tsrc, *parts)
